# baseline (speedup 1.0000x reference)
import functools

import jax
import jax.numpy as jnp
from jax import lax
from jax.experimental import pallas as pl
from jax.experimental.pallas import tpu as pltpu
from jax.experimental.pallas import tpu_sc as plsc

VOCAB = 1000000
CONTEXT = 200
EMBED = 64
BATCH = 4096
SEQ = 200

B = BATCH * SEQ
NC, NS = 2, 16
NW = NC * NS
RPW = B // NW
GB = 100
CHUNK = 200
NCHUNK = RPW // CHUNK
NIDX = RPW // GB


def _sc_body(idx_hbm, tok_hbm, pos_hbm, out_hbm, idx_v, pos_v, gbuf, sem0, sem1):
    sems = (sem0, sem1)
    wid = lax.axis_index("s") * NC + lax.axis_index("c")

    pltpu.sync_copy(idx_hbm.at[wid], idx_v)
    pltpu.sync_copy(pos_hbm, pos_v)

    def fire(c, b):
        for h in range(2):
            pltpu.async_copy(
                tok_hbm.at[idx_v.at[2 * c + h]],
                gbuf.at[b, pl.ds(h * GB, GB)],
                sems[b],
            )

    def drain(b):
        pltpu.make_async_copy(
            out_hbm.at[pl.ds(0, CHUNK)], gbuf.at[b], sems[b]
        ).wait()

    fire(0, 0)
    fire(1, 1)

    def step(i, carry):
        for b in range(2):
            c = 2 * i + b
            drain(b)

            def add_rows(r4, _):
                for q in range(4):
                    r = r4 * 4 + q
                    for k in range(EMBED // 16):
                        sl = pl.ds(k * 16, 16)
                        plsc.addupdate(gbuf.at[b, r, sl], pos_v[r, sl])
                return 0

            lax.fori_loop(0, CHUNK // 4, add_rows, 0)

            row0 = wid * RPW + c * CHUNK
            pltpu.sync_copy(gbuf.at[b], out_hbm.at[pl.ds(row0, CHUNK)])

            @pl.when(c + 2 < NCHUNK)
            def _():
                fire(c + 2, b)

        return carry

    lax.fori_loop(0, NCHUNK // 2, step, 0)


@jax.jit
def _tok_pos_embed(idx3, token_table, position_table):
    mesh = plsc.VectorSubcoreMesh(core_axis_name="c", subcore_axis_name="s")
    f = functools.partial(
        pl.kernel,
        out_type=jax.ShapeDtypeStruct((B, EMBED), jnp.float32),
        mesh=mesh,
        compiler_params=pltpu.CompilerParams(use_tc_tiling_on_sc=False),
        scratch_types=[
            pltpu.VMEM((NIDX, GB), jnp.int32),
            pltpu.VMEM((CONTEXT, EMBED), jnp.float32),
            pltpu.VMEM((2, CHUNK, EMBED), jnp.float32),
            pltpu.SemaphoreType.DMA,
            pltpu.SemaphoreType.DMA,
        ],
    )(_sc_body)
    return f(idx3, token_table, position_table)


def kernel(inputs, token_table, position_table):
    idx3 = inputs.astype(jnp.int32).reshape(NW, NIDX, GB)
    out = _tok_pos_embed(idx3, token_table, position_table)
    return out.reshape(BATCH, SEQ, EMBED)


# speedup vs baseline: 1.0351x; 1.0351x over previous
import functools

import jax
import jax.numpy as jnp
from jax import lax
from jax.experimental import pallas as pl
from jax.experimental.pallas import tpu as pltpu
from jax.experimental.pallas import tpu_sc as plsc

VOCAB = 1000000
CONTEXT = 200
EMBED = 64
BATCH = 4096
SEQ = 200

B = BATCH * SEQ
NC, NS = 2, 16
NW = NC * NS
RPW = B // NW
CHUNK = 100
NCHUNK = RPW // CHUNK
NBUF = 8
DEPTH = 5


def _sc_body(idx_hbm, tok_hbm, pos_hbm, out_hbm,
             idx_v, pos_v, gbuf,
             g0, g1, g2, g3, g4, g5, g6, g7,
             s0, s1, s2, s3, s4, s5, s6, s7):
    gsems = (g0, g1, g2, g3, g4, g5, g6, g7)
    ssems = (s0, s1, s2, s3, s4, s5, s6, s7)
    wid = lax.axis_index("s") * NC + lax.axis_index("c")

    pltpu.sync_copy(idx_hbm.at[wid], idx_v)
    pltpu.sync_copy(pos_hbm, pos_v)

    def fire_gather(c, b):
        pltpu.async_copy(tok_hbm.at[idx_v.at[c]], gbuf.at[b], gsems[b])

    def fire_store(c, b):
        row0 = wid * RPW + c * CHUNK
        pltpu.async_copy(gbuf.at[b], out_hbm.at[pl.ds(row0, CHUNK)], ssems[b])

    def drain(sem, b):
        pltpu.make_async_copy(
            out_hbm.at[pl.ds(0, CHUNK)], gbuf.at[b], sem
        ).wait()

    for c0 in range(DEPTH):
        fire_gather(c0, c0)

    def step(i, carry):
        for b in range(NBUF):
            c = NBUF * i + b
            drain(gsems[b], b)

            par = b % 2

            def add_rows(r4, _):
                for q in range(4):
                    r = r4 * 4 + q
                    for k in range(EMBED // 16):
                        sl = pl.ds(k * 16, 16)
                        plsc.addupdate(gbuf.at[b, r, sl], pos_v[par, r, sl])
                return 0

            lax.fori_loop(0, CHUNK // 4, add_rows, 0)

            fire_store(c, b)

            @pl.when(c + DEPTH < NCHUNK)
            def _():
                bd = (b + DEPTH) % NBUF

                @pl.when(c + DEPTH >= NBUF)
                def _():
                    drain(ssems[bd], bd)

                fire_gather(c + DEPTH, bd)

        return carry

    lax.fori_loop(0, NCHUNK // NBUF, step, 0)

    for b in range(NBUF):
        drain(ssems[b], b)


@jax.jit
def _tok_pos_embed(idx3, token_table, pos2):
    mesh = plsc.VectorSubcoreMesh(core_axis_name="c", subcore_axis_name="s")
    f = functools.partial(
        pl.kernel,
        out_type=jax.ShapeDtypeStruct((B, EMBED), jnp.float32),
        mesh=mesh,
        compiler_params=pltpu.CompilerParams(use_tc_tiling_on_sc=False),
        scratch_types=[
            pltpu.VMEM((NCHUNK, CHUNK), jnp.int32),
            pltpu.VMEM((2, CHUNK, EMBED), jnp.float32),
            pltpu.VMEM((NBUF, CHUNK, EMBED), jnp.float32),
        ] + [pltpu.SemaphoreType.DMA] * (2 * NBUF),
    )(_sc_body)
    return f(idx3, token_table, pos2)


def kernel(inputs, token_table, position_table):
    idx3 = inputs.astype(jnp.int32).reshape(NW, NCHUNK, CHUNK)
    pos2 = position_table.reshape(2, CHUNK, EMBED)
    out = _tok_pos_embed(idx3, token_table, pos2)
    return out.reshape(BATCH, SEQ, EMBED)
